# Initial kernel scaffold; baseline (speedup 1.0000x reference)
import jax
import jax.numpy as jnp
from jax.experimental import pallas as pl


def kernel(tokens, embedding_table):
    raise NotImplementedError("write your pallas kernel here")


# speedup vs baseline: 3.3578x; 3.3578x over previous
import functools

import jax
import jax.numpy as jnp
from jax import lax
from jax.experimental import pallas as pl
from jax.experimental.pallas import tpu as pltpu
from jax.experimental.pallas import tpu_sc as plsc

_VOCAB = 100000
_D = 64
_L = 200
_B = 4096
_N = _B * _L
_NC = 2
_NS = 16
_NW = _NC * _NS
_PER_W = _N // _NW
_CHUNK = 400
_NCHUNK = _PER_W // _CHUNK
_STREAM = 80
_NSTREAM = _CHUNK // _STREAM


def _positional_encoding():
    even_i = jnp.arange(0, _D, 2).astype(jnp.float32)
    denominator = jnp.power(10000.0, even_i / _D)
    position = jnp.arange(_L, dtype=jnp.float32).reshape(_L, 1)
    even_pe = jnp.sin(position / denominator)
    odd_pe = jnp.cos(position / denominator)
    stacked = jnp.stack([even_pe, odd_pe], axis=2)
    return stacked.reshape(_L, _D)


def _body(tok_hbm, table_hbm, pos_hbm, out_hbm, idx_v, rows_v, pos_v, sem):
    wid = lax.axis_index("s") * _NC + lax.axis_index("c")
    base = wid * _PER_W
    pltpu.sync_copy(pos_hbm, pos_v)

    def chunk(g, carry):
        off = base + g * _CHUNK
        pltpu.sync_copy(tok_hbm.at[pl.ds(off, _CHUNK)], idx_v)
        copies = [
            pltpu.async_copy(
                table_hbm.at[idx_v.at[pl.ds(j * _STREAM, _STREAM)]],
                rows_v.at[pl.ds(j * _STREAM, _STREAM)],
                sem,
            )
            for j in range(_NSTREAM)
        ]
        for c in copies:
            c.wait()

        def row(i, c):
            for j in range(4):
                s = pl.ds(j * 16, 16)
                rows_v[i, s] += pos_v[i, s]
            return c

        lax.fori_loop(0, _CHUNK, row, carry)
        pltpu.sync_copy(rows_v, out_hbm.at[pl.ds(off, _CHUNK)])
        return carry

    lax.fori_loop(0, _NCHUNK, chunk, 0)


@functools.partial(jax.jit, static_argnames=())
def kernel(tokens, embedding_table):
    tok = tokens.reshape(-1).astype(jnp.int32)
    pos = jnp.tile(_positional_encoding(), (_CHUNK // _L, 1))
    mesh = plsc.VectorSubcoreMesh(core_axis_name="c", subcore_axis_name="s")
    call = functools.partial(
        pl.kernel,
        out_type=jax.ShapeDtypeStruct((_N, _D), jnp.float32),
        mesh=mesh,
        compiler_params=pltpu.CompilerParams(use_tc_tiling_on_sc=False),
        scratch_types=[
            pltpu.VMEM((_CHUNK,), jnp.int32),
            pltpu.VMEM((_CHUNK, _D), jnp.float32),
            pltpu.VMEM((_CHUNK, _D), jnp.float32),
            pltpu.SemaphoreType.DMA,
        ],
    )(_body)
    out = call(tok, embedding_table, pos)
    return out.reshape(_B, _L, _D)


# speedup vs baseline: 4.0735x; 1.2131x over previous
import functools

import jax
import jax.numpy as jnp
from jax import lax
from jax.experimental import pallas as pl
from jax.experimental.pallas import tpu as pltpu
from jax.experimental.pallas import tpu_sc as plsc

_VOCAB = 100000
_D = 64
_L = 200
_B = 4096
_N = _B * _L
_NC = 2
_NS = 16
_NW = _NC * _NS
_PER_W = _N // _NW
_CHUNK = 400
_NCHUNK = _PER_W // _CHUNK
_STREAM = 80
_NSTREAM = _CHUNK // _STREAM
_NBUF = 4


def _positional_encoding():
    even_i = jnp.arange(0, _D, 2).astype(jnp.float32)
    denominator = jnp.power(10000.0, even_i / _D)
    position = jnp.arange(_L, dtype=jnp.float32).reshape(_L, 1)
    even_pe = jnp.sin(position / denominator)
    odd_pe = jnp.cos(position / denominator)
    stacked = jnp.stack([even_pe, odd_pe], axis=2)
    return stacked.reshape(_L, _D)


def _body(tok_hbm, table_hbm, pos_hbm, out_hbm, idx_v, rows_v, pos_v,
          gsems, osems):
    wid = lax.axis_index("s") * _NC + lax.axis_index("c")
    base = wid * _PER_W
    pltpu.sync_copy(pos_hbm, pos_v)

    def fire_gather(g, b):
        off = base + g * _CHUNK
        pltpu.sync_copy(tok_hbm.at[pl.ds(off, _CHUNK)], idx_v[b])
        for j in range(_NSTREAM):
            pltpu.async_copy(
                table_hbm.at[idx_v[b].at[pl.ds(j * _STREAM, _STREAM)]],
                rows_v[b].at[pl.ds(j * _STREAM, _STREAM)],
                gsems[b],
            )

    def drain_gather(b):
        for j in range(_NSTREAM):
            pltpu.make_async_copy(
                table_hbm.at[idx_v[b].at[pl.ds(j * _STREAM, _STREAM)]],
                rows_v[b].at[pl.ds(j * _STREAM, _STREAM)],
                gsems[b],
            ).wait()

    def add_pos(b):
        rv = rows_v[b]

        @plsc.parallel_loop(0, _L, 1, unroll=8)
        def _add(i):
            for s in range(_CHUNK // _L):
                r = s * _L + i
                for j in range(4):
                    sl = pl.ds(j * 16, 16)
                    rv[r, sl] += pos_v[i, sl]

    def fire_out(g, b):
        off = base + g * _CHUNK
        pltpu.async_copy(rows_v[b], out_hbm.at[pl.ds(off, _CHUNK)], osems[b])

    def drain_out(g, b):
        off = base + g * _CHUNK
        pltpu.make_async_copy(
            rows_v[b], out_hbm.at[pl.ds(off, _CHUNK)], osems[b]
        ).wait()

    def steady(g, b, fire_next=True, wait_prev=True):
        nb = (b + 1) % _NBUF
        if wait_prev:
            drain_out(g - (_NBUF - 1), nb)
        if fire_next:
            fire_gather(g + 1, nb)
        drain_gather(b)
        add_pos(b)
        fire_out(g, b)

    fire_gather(0, 0)
    for g in range(_NBUF - 1):
        steady(g, g % _NBUF, wait_prev=False)
    steady(_NBUF - 1, (_NBUF - 1) % _NBUF)

    @pl.loop(_NBUF, _NCHUNK - _NBUF, step=_NBUF)
    def _loop(g0):
        for b in range(_NBUF):
            steady(g0 + b, b)

    for g in range(_NCHUNK - _NBUF, _NCHUNK - 1):
        steady(g, g % _NBUF)
    steady(_NCHUNK - 1, (_NCHUNK - 1) % _NBUF, fire_next=False)
    for g in range(_NCHUNK - _NBUF + 1, _NCHUNK):
        drain_out(g, g % _NBUF)


@functools.partial(jax.jit, static_argnames=())
def kernel(tokens, embedding_table):
    tok = tokens.reshape(-1).astype(jnp.int32)
    pos = _positional_encoding()
    mesh = plsc.VectorSubcoreMesh(core_axis_name="c", subcore_axis_name="s")
    call = functools.partial(
        pl.kernel,
        out_type=jax.ShapeDtypeStruct((_N, _D), jnp.float32),
        mesh=mesh,
        compiler_params=pltpu.CompilerParams(use_tc_tiling_on_sc=False),
        scratch_types=[
            [pltpu.VMEM((_CHUNK,), jnp.int32) for _ in range(_NBUF)],
            [pltpu.VMEM((_CHUNK, _D), jnp.float32) for _ in range(_NBUF)],
            pltpu.VMEM((_L, _D), jnp.float32),
            [pltpu.SemaphoreType.DMA for _ in range(_NBUF)],
            [pltpu.SemaphoreType.DMA for _ in range(_NBUF)],
        ],
    )(_body)
    out = call(tok, embedding_table, pos)
    return out.reshape(_B, _L, _D)


# speedup vs baseline: 4.0782x; 1.0012x over previous
import functools

import jax
import jax.numpy as jnp
from jax import lax
from jax.experimental import pallas as pl
from jax.experimental.pallas import tpu as pltpu
from jax.experimental.pallas import tpu_sc as plsc

_VOCAB = 100000
_D = 64
_L = 200
_B = 4096
_NC = 2
_NS = 16
_NW = _NC * _NS
_SENT_W = _B // _NW
_SC = 2
_NCHUNK = _SENT_W // _SC
_STREAM = 40
_NSTREAM = _L // _STREAM
_NBUF = 4


def _positional_encoding():
    even_i = jnp.arange(0, _D, 2).astype(jnp.float32)
    denominator = jnp.power(10000.0, even_i / _D)
    position = jnp.arange(_L, dtype=jnp.float32).reshape(_L, 1)
    even_pe = jnp.sin(position / denominator)
    odd_pe = jnp.cos(position / denominator)
    stacked = jnp.stack([even_pe, odd_pe], axis=2)
    return stacked.reshape(_L, _D)


def _body(tok_hbm, table_hbm, pos_hbm, out_hbm, idx_v, rows_v, pos_v,
          gsems, osems):
    wid = lax.axis_index("s") * _NC + lax.axis_index("c")
    base = wid * _SENT_W
    pltpu.sync_copy(pos_hbm, pos_v)

    def fire_gather(g, b):
        off = base + g * _SC
        pltpu.sync_copy(tok_hbm.at[pl.ds(off, _SC)], idx_v[b])
        for s in range(_SC):
            for j in range(_NSTREAM):
                pltpu.async_copy(
                    table_hbm.at[idx_v[b].at[s, pl.ds(j * _STREAM, _STREAM)]],
                    rows_v[b].at[s, pl.ds(j * _STREAM, _STREAM)],
                    gsems[b],
                )

    def drain_gather(b):
        for s in range(_SC):
            for j in range(_NSTREAM):
                pltpu.make_async_copy(
                    table_hbm.at[idx_v[b].at[s, pl.ds(j * _STREAM, _STREAM)]],
                    rows_v[b].at[s, pl.ds(j * _STREAM, _STREAM)],
                    gsems[b],
                ).wait()

    def add_pos(b):
        rv = rows_v[b]

        @plsc.parallel_loop(0, _L, 1, unroll=8)
        def _add(i):
            for s in range(_SC):
                for j in range(4):
                    sl = pl.ds(j * 16, 16)
                    rv[s, i, sl] += pos_v[i, sl]

    def fire_out(g, b):
        off = base + g * _SC
        pltpu.async_copy(rows_v[b], out_hbm.at[pl.ds(off, _SC)], osems[b])

    def drain_out(g, b):
        off = base + g * _SC
        pltpu.make_async_copy(
            rows_v[b], out_hbm.at[pl.ds(off, _SC)], osems[b]
        ).wait()

    def steady(g, b, fire_next=True, wait_prev=True):
        nb = (b + 1) % _NBUF
        if wait_prev:
            drain_out(g - (_NBUF - 1), nb)
        if fire_next:
            fire_gather(g + 1, nb)
        drain_gather(b)
        add_pos(b)
        fire_out(g, b)

    fire_gather(0, 0)
    for g in range(_NBUF - 1):
        steady(g, g % _NBUF, wait_prev=False)
    steady(_NBUF - 1, (_NBUF - 1) % _NBUF)

    @pl.loop(_NBUF, _NCHUNK - _NBUF, step=_NBUF)
    def _loop(g0):
        for b in range(_NBUF):
            steady(g0 + b, b)

    for g in range(_NCHUNK - _NBUF, _NCHUNK - 1):
        steady(g, g % _NBUF)
    steady(_NCHUNK - 1, (_NCHUNK - 1) % _NBUF, fire_next=False)
    for g in range(_NCHUNK - _NBUF + 1, _NCHUNK):
        drain_out(g, g % _NBUF)


@functools.partial(jax.jit, static_argnames=())
def kernel(tokens, embedding_table):
    tok = tokens.astype(jnp.int32)
    pos = _positional_encoding()
    mesh = plsc.VectorSubcoreMesh(core_axis_name="c", subcore_axis_name="s")
    call = functools.partial(
        pl.kernel,
        out_type=jax.ShapeDtypeStruct((_B, _L, _D), jnp.float32),
        mesh=mesh,
        compiler_params=pltpu.CompilerParams(use_tc_tiling_on_sc=False),
        scratch_types=[
            [pltpu.VMEM((_SC, _L), jnp.int32) for _ in range(_NBUF)],
            [pltpu.VMEM((_SC, _L, _D), jnp.float32) for _ in range(_NBUF)],
            pltpu.VMEM((_L, _D), jnp.float32),
            [pltpu.SemaphoreType.DMA for _ in range(_NBUF)],
            [pltpu.SemaphoreType.DMA for _ in range(_NBUF)],
        ],
    )(_body)
    return call(tok, embedding_table, pos)
